# baseline (speedup 1.0000x reference)
import functools

import jax
import jax.numpy as jnp
from jax import lax
from jax.experimental import pallas as pl
from jax.experimental.pallas import tpu as pltpu
from jax.experimental.pallas import tpu_sc as plsc

B = 16384
V = 100000
D = 317
H1, H2, H3 = 1024, 512, 256
OUT_D = D + H3


_NC = 2
_NS = 16
_NW = _NC * _NS
_B_PER_W = B // _NW
_CHUNK = 128
_N_CHUNKS = _B_PER_W // _CHUNK
_DP = 384

_sc_mesh = plsc.VectorSubcoreMesh(core_axis_name="c", subcore_axis_name="s")


@functools.partial(
    pl.kernel,
    mesh=_sc_mesh,
    out_type=jax.ShapeDtypeStruct((B, _DP), jnp.float32),
    scratch_types=[
        pltpu.VMEM((_B_PER_W,), jnp.int32),
        pltpu.VMEM((2, _CHUNK, _DP), jnp.float32),
        pltpu.SemaphoreType.DMA,
        pltpu.SemaphoreType.DMA,
    ],
)
def _sc_gather(table_hbm, tail_hbm, idx_hbm, out_hbm, idx_v, rows_v, gsem, wsem):
    wid = lax.axis_index("s") * _NC + lax.axis_index("c")
    base = wid * _B_PER_W
    pltpu.sync_copy(idx_hbm.at[pl.ds(base, _B_PER_W)], idx_v)

    def gather(c, slot):
        ids = idx_v.at[pl.ds(c * _CHUNK, _CHUNK)]
        pltpu.async_copy(table_hbm.at[ids, pl.ds(0, 128)],
                         rows_v.at[slot, :, pl.ds(0, 128)], gsem)
        pltpu.async_copy(table_hbm.at[ids, pl.ds(128, 128)],
                         rows_v.at[slot, :, pl.ds(128, 128)], gsem)
        pltpu.async_copy(tail_hbm.at[ids],
                         rows_v.at[slot, :, pl.ds(256, 128)], gsem)

    def put(c, slot):
        pltpu.async_copy(rows_v.at[slot],
                         out_hbm.at[pl.ds(base + c * _CHUNK, _CHUNK)], wsem)

    gather(0, 0)
    for c in range(_N_CHUNKS):
        slot = c % 2
        pltpu.make_async_copy(table_hbm, rows_v.at[slot], gsem).wait()
        if c >= 1:
            pltpu.make_async_copy(rows_v.at[0], out_hbm.at[pl.ds(0, _CHUNK)],
                                  wsem).wait()
        if c + 1 < _N_CHUNKS:
            gather(c + 1, 1 - slot)
        put(c, slot)
    pltpu.make_async_copy(rows_v.at[0], out_hbm.at[pl.ds(0, _CHUNK)],
                          wsem).wait()


_RB = 2000
_TAIL = D - 256


def _tail_body(t_ref, out_ref, vacc, sem):
    i = pl.program_id(0)
    cp = pltpu.make_async_copy(
        t_ref.at[pl.ds(i * _RB, _RB), pl.ds(256, _TAIL)], vacc, sem)
    cp.start()
    out_ref[:, _TAIL:] = jnp.zeros((_RB, 128 - _TAIL), jnp.float32)
    cp.wait()
    out_ref[:, :_TAIL] = vacc[...]


_tail_call = pl.pallas_call(
    _tail_body,
    grid=(V // _RB,),
    in_specs=[pl.BlockSpec(memory_space=pl.ANY)],
    out_specs=pl.BlockSpec((_RB, 128), lambda i: (i, 0)),
    out_shape=jax.ShapeDtypeStruct((V, 128), jnp.float32),
    scratch_shapes=[pltpu.VMEM((_RB, _TAIL), jnp.float32),
                    pltpu.SemaphoreType.DMA],
    compiler_params=pltpu.CompilerParams(
        dimension_semantics=("arbitrary",),
    ),
)


_TB = 1024


def _mlp_body(feats_ref, emb_ref, w1_ref, b1_ref, w2_ref, b2_ref, w3_ref,
              b3_ref, out_ref):
    f = feats_ref[...]
    h = jnp.dot(f, w1_ref[...], preferred_element_type=jnp.float32)
    h = jnp.maximum(h + b1_ref[...], 0.0)
    h = jnp.dot(h.astype(jnp.bfloat16), w2_ref[...],
                preferred_element_type=jnp.float32)
    h = jnp.maximum(h + b2_ref[...], 0.0)
    z = jnp.dot(h.astype(jnp.bfloat16), w3_ref[...],
                preferred_element_type=jnp.float32)
    z = z + b3_ref[...]
    z = z - jnp.max(z, axis=-1, keepdims=True)
    e = jnp.exp(z)
    sm = e / jnp.sum(e, axis=-1, keepdims=True)
    out_ref[:, :D] = emb_ref[:, :D]
    out_ref[:, D:] = sm


_mlp_call = pl.pallas_call(
    _mlp_body,
    grid=(B // _TB,),
    in_specs=[
        pl.BlockSpec((_TB, 8), lambda i: (i, 0)),
        pl.BlockSpec((_TB, _DP), lambda i: (i, 0)),
        pl.BlockSpec((8, H1), lambda i: (0, 0)),
        pl.BlockSpec((1, H1), lambda i: (0, 0)),
        pl.BlockSpec((H1, H2), lambda i: (0, 0)),
        pl.BlockSpec((1, H2), lambda i: (0, 0)),
        pl.BlockSpec((H2, H3), lambda i: (0, 0)),
        pl.BlockSpec((1, H3), lambda i: (0, 0)),
    ],
    out_specs=pl.BlockSpec((_TB, OUT_D), lambda i: (i, 0)),
    out_shape=jax.ShapeDtypeStruct((B, OUT_D), jnp.float32),
    compiler_params=pltpu.CompilerParams(
        dimension_semantics=("arbitrary",),
    ),
)


def kernel(anime_id, genre, type, episodes, general_rating, members, user_id,
           user_rating, table, W1, b1, W2, b2, W3, b3):
    idx = genre.astype(jnp.int32)
    tail = _tail_call(table)
    emb = _sc_gather(table, tail, idx)

    feats = jnp.stack(
        [anime_id, type, episodes, general_rating, members, user_id,
         user_rating], axis=-1)
    feats = jnp.pad(feats, ((0, 0), (0, 1)))
    w1p = jnp.pad(W1, ((0, 1), (0, 0)))

    out = _mlp_call(feats, emb, w1p, b1.reshape(1, H1),
                    W2.astype(jnp.bfloat16), b2.reshape(1, H2),
                    W3.astype(jnp.bfloat16), b3.reshape(1, H3))
    return out


# speedup vs baseline: 1.0110x; 1.0110x over previous
import functools

import jax
import jax.numpy as jnp
from jax import lax
from jax.experimental import pallas as pl
from jax.experimental.pallas import tpu as pltpu
from jax.experimental.pallas import tpu_sc as plsc

B = 16384
V = 100000
D = 317
H1, H2, H3 = 1024, 512, 256
OUT_D = D + H3


_NC = 2
_NS = 16
_NW = _NC * _NS
_B_PER_W = B // _NW
_CHUNK = 128
_N_CHUNKS = _B_PER_W // _CHUNK
_DP = 384

_sc_mesh = plsc.VectorSubcoreMesh(core_axis_name="c", subcore_axis_name="s")


@functools.partial(
    pl.kernel,
    mesh=_sc_mesh,
    out_type=jax.ShapeDtypeStruct((B, _DP), jnp.float32),
    scratch_types=[
        pltpu.VMEM((_B_PER_W,), jnp.int32),
        pltpu.VMEM((2, _CHUNK, _DP), jnp.float32),
        pltpu.SemaphoreType.DMA,
        pltpu.SemaphoreType.DMA,
    ],
)
def _sc_gather(table_hbm, tail_hbm, idx_hbm, out_hbm, idx_v, rows_v, gsem, wsem):
    wid = lax.axis_index("s") * _NC + lax.axis_index("c")
    base = wid * _B_PER_W
    pltpu.sync_copy(idx_hbm.at[pl.ds(base, _B_PER_W)], idx_v)

    def gather(c, slot):
        ids = idx_v.at[pl.ds(c * _CHUNK, _CHUNK)]
        pltpu.async_copy(table_hbm.at[ids, pl.ds(0, 128)],
                         rows_v.at[slot, :, pl.ds(0, 128)], gsem)
        pltpu.async_copy(table_hbm.at[ids, pl.ds(128, 128)],
                         rows_v.at[slot, :, pl.ds(128, 128)], gsem)
        pltpu.async_copy(tail_hbm.at[ids],
                         rows_v.at[slot, :, pl.ds(256, 128)], gsem)

    def put(c, slot):
        pltpu.async_copy(rows_v.at[slot],
                         out_hbm.at[pl.ds(base + c * _CHUNK, _CHUNK)], wsem)

    gather(0, 0)
    for c in range(_N_CHUNKS):
        slot = c % 2
        pltpu.make_async_copy(table_hbm, rows_v.at[slot], gsem).wait()
        if c >= 1:
            pltpu.make_async_copy(rows_v.at[0], out_hbm.at[pl.ds(0, _CHUNK)],
                                  wsem).wait()
        if c + 1 < _N_CHUNKS:
            gather(c + 1, 1 - slot)
        put(c, slot)
    pltpu.make_async_copy(rows_v.at[0], out_hbm.at[pl.ds(0, _CHUNK)],
                          wsem).wait()


_RB = 2000
_TAIL = D - 256


def _tail_body(t_ref, out_ref, vacc, sem):
    i = pl.program_id(0)
    cp = pltpu.make_async_copy(
        t_ref.at[pl.ds(i * _RB, _RB), pl.ds(256, _TAIL)], vacc, sem)
    cp.start()
    out_ref[:, _TAIL:] = jnp.zeros((_RB, 128 - _TAIL), jnp.float32)
    cp.wait()
    out_ref[:, :_TAIL] = vacc[...]


_tail_call = pl.pallas_call(
    _tail_body,
    grid=(V // _RB,),
    in_specs=[pl.BlockSpec(memory_space=pl.ANY)],
    out_specs=pl.BlockSpec((_RB, 128), lambda i: (i, 0)),
    out_shape=jax.ShapeDtypeStruct((V, 128), jnp.float32),
    scratch_shapes=[pltpu.VMEM((_RB, _TAIL), jnp.float32),
                    pltpu.SemaphoreType.DMA],
    compiler_params=pltpu.CompilerParams(
        dimension_semantics=("arbitrary",),
    ),
)


_TB = 1024


def _mlp_body(feats_ref, emb_ref, w1_ref, b1_ref, w2_ref, b2_ref, w3_ref,
              b3_ref, out_ref):
    f = feats_ref[...]
    h = jnp.dot(f, w1_ref[...], preferred_element_type=jnp.float32)
    h = jnp.maximum(h + b1_ref[...], 0.0)
    h = jnp.dot(h.astype(jnp.bfloat16), w2_ref[...],
                preferred_element_type=jnp.float32)
    h = jnp.maximum(h + b2_ref[...], 0.0)
    z = jnp.dot(h.astype(jnp.bfloat16), w3_ref[...],
                preferred_element_type=jnp.float32)
    z = z + b3_ref[...]
    z = z - jnp.max(z, axis=-1, keepdims=True)
    e = jnp.exp(z)
    sm = e / jnp.sum(e, axis=-1, keepdims=True)
    out_ref[:, :D] = emb_ref[:, :D]
    out_ref[:, D:] = sm


_mlp_call = pl.pallas_call(
    _mlp_body,
    grid=(B // _TB,),
    in_specs=[
        pl.BlockSpec((_TB, 7), lambda i: (i, 0)),
        pl.BlockSpec((_TB, _DP), lambda i: (i, 0)),
        pl.BlockSpec((7, H1), lambda i: (0, 0)),
        pl.BlockSpec((1, H1), lambda i: (0, 0)),
        pl.BlockSpec((H1, H2), lambda i: (0, 0)),
        pl.BlockSpec((1, H2), lambda i: (0, 0)),
        pl.BlockSpec((H2, H3), lambda i: (0, 0)),
        pl.BlockSpec((1, H3), lambda i: (0, 0)),
    ],
    out_specs=pl.BlockSpec((_TB, OUT_D), lambda i: (i, 0)),
    out_shape=jax.ShapeDtypeStruct((B, OUT_D), jnp.float32),
    compiler_params=pltpu.CompilerParams(
        dimension_semantics=("arbitrary",),
    ),
)


def kernel(anime_id, genre, type, episodes, general_rating, members, user_id,
           user_rating, table, W1, b1, W2, b2, W3, b3):
    idx = genre.astype(jnp.int32)
    feats = jnp.stack(
        [anime_id, type, episodes, general_rating, members, user_id,
         user_rating], axis=-1)
    w2b = W2.astype(jnp.bfloat16)
    w3b = W3.astype(jnp.bfloat16)

    tail = _tail_call(table)
    emb = _sc_gather(table, tail, idx)

    out = _mlp_call(feats, emb, W1, b1.reshape(1, H1),
                    w2b, b2.reshape(1, H2),
                    w3b, b3.reshape(1, H3))
    return out


# speedup vs baseline: 1.1404x; 1.1279x over previous
import functools

import jax
import jax.numpy as jnp
from jax import lax
from jax.experimental import pallas as pl
from jax.experimental.pallas import tpu as pltpu
from jax.experimental.pallas import tpu_sc as plsc

B = 16384
V = 100000
D = 317
H1, H2, H3 = 1024, 512, 256
OUT_D = D + H3


_NC = 2
_NS = 16
_NW = _NC * _NS
_B_PER_W = B // _NW
_CHUNK = 128
_N_CHUNKS = _B_PER_W // _CHUNK
_DP = 384

_sc_mesh = plsc.VectorSubcoreMesh(core_axis_name="c", subcore_axis_name="s")


@functools.partial(
    pl.kernel,
    mesh=_sc_mesh,
    out_type=jax.ShapeDtypeStruct((B, _DP), jnp.float32),
    scratch_types=[
        pltpu.VMEM((_B_PER_W,), jnp.int32),
        pltpu.VMEM((2, _CHUNK, _DP), jnp.float32),
        pltpu.SemaphoreType.DMA,
        pltpu.SemaphoreType.DMA,
    ],
)
def _sc_gather(table_hbm, tail_hbm, idx_hbm, out_hbm, idx_v, rows_v, gsem, wsem):
    wid = lax.axis_index("s") * _NC + lax.axis_index("c")
    base = wid * _B_PER_W
    pltpu.sync_copy(idx_hbm.at[pl.ds(base, _B_PER_W)], idx_v)

    def gather(c, slot):
        ids = idx_v.at[pl.ds(c * _CHUNK, _CHUNK)]
        pltpu.async_copy(table_hbm.at[ids, pl.ds(0, 128)],
                         rows_v.at[slot, :, pl.ds(0, 128)], gsem)
        pltpu.async_copy(table_hbm.at[ids, pl.ds(128, 128)],
                         rows_v.at[slot, :, pl.ds(128, 128)], gsem)
        pltpu.async_copy(tail_hbm.at[ids],
                         rows_v.at[slot, :, pl.ds(256, 128)], gsem)

    def put(c, slot):
        pltpu.async_copy(rows_v.at[slot],
                         out_hbm.at[pl.ds(base + c * _CHUNK, _CHUNK)], wsem)

    gather(0, 0)
    for c in range(_N_CHUNKS):
        slot = c % 2
        pltpu.make_async_copy(table_hbm, rows_v.at[slot], gsem).wait()
        if c >= 1:
            pltpu.make_async_copy(rows_v.at[0], out_hbm.at[pl.ds(0, _CHUNK)],
                                  wsem).wait()
        if c + 1 < _N_CHUNKS:
            gather(c + 1, 1 - slot)
        put(c, slot)
    pltpu.make_async_copy(rows_v.at[0], out_hbm.at[pl.ds(0, _CHUNK)],
                          wsem).wait()


_RB = 2000


def _tail_body(t_ref, out_ref):
    out_ref[...] = t_ref[...]


_tail_call = pl.pallas_call(
    _tail_body,
    grid=(V // _RB,),
    in_specs=[pl.BlockSpec((_RB, 128), lambda i: (i, 2))],
    out_specs=pl.BlockSpec((_RB, 128), lambda i: (i, 0)),
    out_shape=jax.ShapeDtypeStruct((V, 128), jnp.float32),
    compiler_params=pltpu.CompilerParams(
        dimension_semantics=("arbitrary",),
    ),
)


_TB = 1024


def _mlp_body(feats_ref, emb_ref, w1_ref, b1_ref, w2_ref, b2_ref, w3_ref,
              b3_ref, out_ref):
    f = feats_ref[...]
    h = jnp.dot(f, w1_ref[...], preferred_element_type=jnp.float32)
    h = jnp.maximum(h + b1_ref[...], 0.0)
    h = jnp.dot(h.astype(jnp.bfloat16), w2_ref[...],
                preferred_element_type=jnp.float32)
    h = jnp.maximum(h + b2_ref[...], 0.0)
    z = jnp.dot(h.astype(jnp.bfloat16), w3_ref[...],
                preferred_element_type=jnp.float32)
    z = z + b3_ref[...]
    z = z - jnp.max(z, axis=-1, keepdims=True)
    e = jnp.exp(z)
    sm = e / jnp.sum(e, axis=-1, keepdims=True)
    out_ref[:, :D] = emb_ref[:, :D]
    out_ref[:, D:] = sm


_mlp_call = pl.pallas_call(
    _mlp_body,
    grid=(B // _TB,),
    in_specs=[
        pl.BlockSpec((_TB, 7), lambda i: (i, 0)),
        pl.BlockSpec((_TB, _DP), lambda i: (i, 0)),
        pl.BlockSpec((7, H1), lambda i: (0, 0)),
        pl.BlockSpec((1, H1), lambda i: (0, 0)),
        pl.BlockSpec((H1, H2), lambda i: (0, 0)),
        pl.BlockSpec((1, H2), lambda i: (0, 0)),
        pl.BlockSpec((H2, H3), lambda i: (0, 0)),
        pl.BlockSpec((1, H3), lambda i: (0, 0)),
    ],
    out_specs=pl.BlockSpec((_TB, OUT_D), lambda i: (i, 0)),
    out_shape=jax.ShapeDtypeStruct((B, OUT_D), jnp.float32),
    compiler_params=pltpu.CompilerParams(
        dimension_semantics=("arbitrary",),
    ),
)


def kernel(anime_id, genre, type, episodes, general_rating, members, user_id,
           user_rating, table, W1, b1, W2, b2, W3, b3):
    idx = genre.astype(jnp.int32)
    feats = jnp.stack(
        [anime_id, type, episodes, general_rating, members, user_id,
         user_rating], axis=-1)
    w2b = W2.astype(jnp.bfloat16)
    w3b = W3.astype(jnp.bfloat16)

    tail = _tail_call(table)
    emb = _sc_gather(table, tail, idx)

    out = _mlp_call(feats, emb, W1, b1.reshape(1, H1),
                    w2b, b2.reshape(1, H2),
                    w3b, b3.reshape(1, H3))
    return out


# speedup vs baseline: 1.5323x; 1.3437x over previous
import functools

import jax
import jax.numpy as jnp
from jax import lax
from jax.experimental import pallas as pl
from jax.experimental.pallas import tpu as pltpu
from jax.experimental.pallas import tpu_sc as plsc

B = 16384
V = 100000
D = 317
H1, H2, H3 = 1024, 512, 256
OUT_D = D + H3


_NC = 2
_NS = 16
_NW = _NC * _NS
_B_PER_W = B // _NW
_CHUNK = 128
_N_CHUNKS = _B_PER_W // _CHUNK
_DP = 384

_sc_mesh = plsc.VectorSubcoreMesh(core_axis_name="c", subcore_axis_name="s")


@functools.partial(
    pl.kernel,
    mesh=_sc_mesh,
    out_type=jax.ShapeDtypeStruct((B, _DP), jnp.float32),
    scratch_types=[
        pltpu.VMEM((_B_PER_W,), jnp.int32),
        pltpu.VMEM((2, _CHUNK, _DP), jnp.float32),
        pltpu.SemaphoreType.DMA,
        pltpu.SemaphoreType.DMA,
    ],
)
def _sc_gather(table_hbm, idx_hbm, out_hbm, idx_v, rows_v, gsem, wsem):
    wid = lax.axis_index("s") * _NC + lax.axis_index("c")
    base = wid * _B_PER_W
    pltpu.sync_copy(idx_hbm.at[pl.ds(base, _B_PER_W)], idx_v)

    def gather(c, slot):
        ids = idx_v.at[pl.ds(c * _CHUNK, _CHUNK)]
        pltpu.async_copy(table_hbm.at[ids], rows_v.at[slot], gsem)

    def put(c, slot):
        pltpu.async_copy(rows_v.at[slot],
                         out_hbm.at[pl.ds(base + c * _CHUNK, _CHUNK)], wsem)

    gather(0, 0)
    for c in range(_N_CHUNKS):
        slot = c % 2
        pltpu.make_async_copy(table_hbm, rows_v.at[slot], gsem).wait()
        if c >= 1:
            pltpu.make_async_copy(rows_v.at[0], out_hbm.at[pl.ds(0, _CHUNK)],
                                  wsem).wait()
        if c + 1 < _N_CHUNKS:
            gather(c + 1, 1 - slot)
        put(c, slot)
    pltpu.make_async_copy(rows_v.at[0], out_hbm.at[pl.ds(0, _CHUNK)],
                          wsem).wait()


_CB = 2048


def _tp_body(t_ref, out_ref):
    out_ref[:, :D] = lax.transpose(t_ref[...], (1, 0))
    out_ref[:, D:] = jnp.zeros((_CB, _DP - D), jnp.float32)


_tp_call = pl.pallas_call(
    _tp_body,
    grid=((V + _CB - 1) // _CB,),
    in_specs=[pl.BlockSpec((D, _CB), lambda i: (0, i))],
    out_specs=pl.BlockSpec((_CB, _DP), lambda i: (i, 0)),
    out_shape=jax.ShapeDtypeStruct((V, _DP), jnp.float32),
    compiler_params=pltpu.CompilerParams(
        dimension_semantics=("arbitrary",),
    ),
)


_TB = 1024


def _mlp_body(feats_ref, emb_ref, w1_ref, b1_ref, w2_ref, b2_ref, w3_ref,
              b3_ref, out_ref):
    f = feats_ref[...]
    h = jnp.dot(f, w1_ref[...], preferred_element_type=jnp.float32)
    h = jnp.maximum(h + b1_ref[...], 0.0)
    h = jnp.dot(h.astype(jnp.bfloat16), w2_ref[...],
                preferred_element_type=jnp.float32)
    h = jnp.maximum(h + b2_ref[...], 0.0)
    z = jnp.dot(h.astype(jnp.bfloat16), w3_ref[...],
                preferred_element_type=jnp.float32)
    z = z + b3_ref[...]
    z = z - jnp.max(z, axis=-1, keepdims=True)
    e = jnp.exp(z)
    sm = e / jnp.sum(e, axis=-1, keepdims=True)
    out_ref[:, :D] = emb_ref[:, :D]
    out_ref[:, D:] = sm


_mlp_call = pl.pallas_call(
    _mlp_body,
    grid=(B // _TB,),
    in_specs=[
        pl.BlockSpec((_TB, 7), lambda i: (i, 0)),
        pl.BlockSpec((_TB, _DP), lambda i: (i, 0)),
        pl.BlockSpec((7, H1), lambda i: (0, 0)),
        pl.BlockSpec((1, H1), lambda i: (0, 0)),
        pl.BlockSpec((H1, H2), lambda i: (0, 0)),
        pl.BlockSpec((1, H2), lambda i: (0, 0)),
        pl.BlockSpec((H2, H3), lambda i: (0, 0)),
        pl.BlockSpec((1, H3), lambda i: (0, 0)),
    ],
    out_specs=pl.BlockSpec((_TB, OUT_D), lambda i: (i, 0)),
    out_shape=jax.ShapeDtypeStruct((B, OUT_D), jnp.float32),
    compiler_params=pltpu.CompilerParams(
        dimension_semantics=("arbitrary",),
    ),
)


def kernel(anime_id, genre, type, episodes, general_rating, members, user_id,
           user_rating, table, W1, b1, W2, b2, W3, b3):
    idx = genre.astype(jnp.int32)
    feats = jnp.stack(
        [anime_id, type, episodes, general_rating, members, user_id,
         user_rating], axis=-1)
    w2b = W2.astype(jnp.bfloat16)
    w3b = W3.astype(jnp.bfloat16)

    table_p = _tp_call(jnp.swapaxes(table, 0, 1))
    emb = _sc_gather(table_p, idx)

    out = _mlp_call(feats, emb, W1, b1.reshape(1, H1),
                    w2b, b2.reshape(1, H2),
                    w3b, b3.reshape(1, H3))
    return out


# speedup vs baseline: 2.0118x; 1.3130x over previous
import functools

import jax
import jax.numpy as jnp
from jax import lax
from jax.experimental import pallas as pl
from jax.experimental.pallas import tpu as pltpu
from jax.experimental.pallas import tpu_sc as plsc

B = 16384
V = 100000
D = 317
H1, H2, H3 = 1024, 512, 256
OUT_D = D + H3


_NC = 2
_NS = 16
_NW = _NC * _NS
_B_PER_W = B // _NW
_CHUNK = 128
_N_CHUNKS = _B_PER_W // _CHUNK
_DP = 384

_sc_mesh = plsc.VectorSubcoreMesh(core_axis_name="c", subcore_axis_name="s")


@functools.partial(
    pl.kernel,
    mesh=_sc_mesh,
    out_type=jax.ShapeDtypeStruct((B, _DP), jnp.float32),
    scratch_types=[
        pltpu.VMEM((_B_PER_W,), jnp.int32),
        pltpu.VMEM((2, _CHUNK, _DP), jnp.float32),
        pltpu.SemaphoreType.DMA,
        pltpu.SemaphoreType.DMA,
    ],
)
def _sc_gather(table_hbm, idx_hbm, out_hbm, idx_v, rows_v, gsem, wsem):
    wid = lax.axis_index("s") * _NC + lax.axis_index("c")
    base = wid * _B_PER_W
    pltpu.sync_copy(idx_hbm.at[pl.ds(base, _B_PER_W)], idx_v)

    def gather(c, slot):
        ids = idx_v.at[pl.ds(c * _CHUNK, _CHUNK)]
        pltpu.async_copy(table_hbm.at[ids], rows_v.at[slot], gsem)

    def put(c, slot):
        pltpu.async_copy(rows_v.at[slot],
                         out_hbm.at[pl.ds(base + c * _CHUNK, _CHUNK)], wsem)

    gather(0, 0)
    for c in range(_N_CHUNKS):
        slot = c % 2
        pltpu.make_async_copy(table_hbm, rows_v.at[slot], gsem).wait()
        if c >= 1:
            pltpu.make_async_copy(rows_v.at[0], out_hbm.at[pl.ds(0, _CHUNK)],
                                  wsem).wait()
        if c + 1 < _N_CHUNKS:
            gather(c + 1, 1 - slot)
        put(c, slot)
    pltpu.make_async_copy(rows_v.at[0], out_hbm.at[pl.ds(0, _CHUNK)],
                          wsem).wait()


_CB = 2048


def _tp_body(t_ref, out_ref):
    out_ref[:, :D] = lax.transpose(t_ref[...], (1, 0))
    out_ref[:, D:] = jnp.zeros((_CB, _DP - D), jnp.float32)


_tp_call = pl.pallas_call(
    _tp_body,
    grid=((V + _CB - 1) // _CB,),
    in_specs=[pl.BlockSpec((D, _CB), lambda i: (0, i))],
    out_specs=pl.BlockSpec((_CB, _DP), lambda i: (i, 0)),
    out_shape=jax.ShapeDtypeStruct((V, _DP), jnp.float32),
    compiler_params=pltpu.CompilerParams(
        dimension_semantics=("arbitrary",),
    ),
)


_TB = 1024


def _mlp_body(featsT_ref, emb_ref, w1t_ref, b1_ref, w2t_ref, b2_ref, w3t_ref,
              b3_ref, out_ref):
    f = featsT_ref[...]
    h = jnp.dot(w1t_ref[...], f, preferred_element_type=jnp.float32)
    h = jnp.maximum(h + b1_ref[...], 0.0)
    h = jnp.dot(w2t_ref[...], h.astype(jnp.bfloat16),
                preferred_element_type=jnp.float32)
    h = jnp.maximum(h + b2_ref[...], 0.0)
    z = jnp.dot(w3t_ref[...], h.astype(jnp.bfloat16),
                preferred_element_type=jnp.float32)
    z = z + b3_ref[...]
    z = z - jnp.max(z, axis=0, keepdims=True)
    e = jnp.exp(z)
    sm = e / jnp.sum(e, axis=0, keepdims=True)
    out_ref[:D, :] = lax.transpose(emb_ref[...], (1, 0))[:D, :]
    out_ref[D:, :] = sm


_mlp_call = pl.pallas_call(
    _mlp_body,
    grid=(B // _TB,),
    in_specs=[
        pl.BlockSpec((7, _TB), lambda i: (0, i)),
        pl.BlockSpec((_TB, _DP), lambda i: (i, 0)),
        pl.BlockSpec((H1, 7), lambda i: (0, 0)),
        pl.BlockSpec((H1, 1), lambda i: (0, 0)),
        pl.BlockSpec((H2, H1), lambda i: (0, 0)),
        pl.BlockSpec((H2, 1), lambda i: (0, 0)),
        pl.BlockSpec((H3, H2), lambda i: (0, 0)),
        pl.BlockSpec((H3, 1), lambda i: (0, 0)),
    ],
    out_specs=pl.BlockSpec((OUT_D, _TB), lambda i: (0, i)),
    out_shape=jax.ShapeDtypeStruct((OUT_D, B), jnp.float32),
    compiler_params=pltpu.CompilerParams(
        dimension_semantics=("arbitrary",),
    ),
)


def kernel(anime_id, genre, type, episodes, general_rating, members, user_id,
           user_rating, table, W1, b1, W2, b2, W3, b3):
    idx = genre.astype(jnp.int32)
    featsT = jnp.stack(
        [anime_id, type, episodes, general_rating, members, user_id,
         user_rating], axis=0)
    w1t = jnp.swapaxes(W1, 0, 1)
    w2t = jnp.swapaxes(W2, 0, 1).astype(jnp.bfloat16)
    w3t = jnp.swapaxes(W3, 0, 1).astype(jnp.bfloat16)

    table_p = _tp_call(jnp.swapaxes(table, 0, 1))
    emb = _sc_gather(table_p, idx)

    out_t = _mlp_call(featsT, emb, w1t, b1.reshape(H1, 1),
                      w2t, b2.reshape(H2, 1),
                      w3t, b3.reshape(H3, 1))
    return jnp.swapaxes(out_t, 0, 1)


# speedup vs baseline: 2.1135x; 1.0506x over previous
import functools

import jax
import jax.numpy as jnp
from jax import lax
from jax.experimental import pallas as pl
from jax.experimental.pallas import tpu as pltpu
from jax.experimental.pallas import tpu_sc as plsc

B = 16384
V = 100000
D = 317
H1, H2, H3 = 1024, 512, 256
OUT_D = D + H3


_NC = 2
_NS = 16
_NW = _NC * _NS
_B_PER_W = B // _NW
_CHUNK = 128
_N_CHUNKS = _B_PER_W // _CHUNK
_DP = 384

_sc_mesh = plsc.VectorSubcoreMesh(core_axis_name="c", subcore_axis_name="s")


@functools.partial(
    pl.kernel,
    mesh=_sc_mesh,
    out_type=jax.ShapeDtypeStruct((B, _DP), jnp.float32),
    scratch_types=[
        pltpu.VMEM((_B_PER_W,), jnp.int32),
        pltpu.VMEM((2, _CHUNK, _DP), jnp.float32),
        pltpu.SemaphoreType.DMA,
        pltpu.SemaphoreType.DMA,
    ],
)
def _sc_gather(table_hbm, idx_hbm, out_hbm, idx_v, rows_v, gsem, wsem):
    wid = lax.axis_index("s") * _NC + lax.axis_index("c")
    base = wid * _B_PER_W
    pltpu.sync_copy(idx_hbm.at[pl.ds(base, _B_PER_W)], idx_v)

    def gather(c, slot):
        ids = idx_v.at[pl.ds(c * _CHUNK, _CHUNK)]
        pltpu.async_copy(table_hbm.at[ids], rows_v.at[slot], gsem)

    def put(c, slot):
        pltpu.async_copy(rows_v.at[slot],
                         out_hbm.at[pl.ds(base + c * _CHUNK, _CHUNK)], wsem)

    gather(0, 0)
    for c in range(_N_CHUNKS):
        slot = c % 2
        pltpu.make_async_copy(table_hbm, rows_v.at[slot], gsem).wait()
        if c >= 1:
            pltpu.make_async_copy(rows_v.at[0], out_hbm.at[pl.ds(0, _CHUNK)],
                                  wsem).wait()
        if c + 1 < _N_CHUNKS:
            gather(c + 1, 1 - slot)
        put(c, slot)
    pltpu.make_async_copy(rows_v.at[0], out_hbm.at[pl.ds(0, _CHUNK)],
                          wsem).wait()


_CB = 4096


def _tp_body(t_ref, out_ref):
    out_ref[:, :D] = lax.transpose(t_ref[...], (1, 0))
    out_ref[:, D:] = jnp.zeros((_CB, _DP - D), jnp.float32)


_tp_call = pl.pallas_call(
    _tp_body,
    grid=((V + _CB - 1) // _CB,),
    in_specs=[pl.BlockSpec((D, _CB), lambda i: (0, i))],
    out_specs=pl.BlockSpec((_CB, _DP), lambda i: (i, 0)),
    out_shape=jax.ShapeDtypeStruct((V, _DP), jnp.float32),
    compiler_params=pltpu.CompilerParams(
        dimension_semantics=("arbitrary",),
    ),
)


_TB = 2048


def _mlp_body(featsT_ref, emb_ref, w1t_ref, b1_ref, w2t_ref, b2_ref, w3t_ref,
              b3_ref, out_ref):
    f = featsT_ref[...]
    h = jnp.dot(w1t_ref[...], f, preferred_element_type=jnp.float32)
    h = jnp.maximum(h + b1_ref[...], 0.0)
    h = jnp.dot(w2t_ref[...], h.astype(jnp.bfloat16),
                preferred_element_type=jnp.float32)
    h = jnp.maximum(h + b2_ref[...], 0.0)
    z = jnp.dot(w3t_ref[...], h.astype(jnp.bfloat16),
                preferred_element_type=jnp.float32)
    z = z + b3_ref[...]
    z = z - jnp.max(z, axis=0, keepdims=True)
    e = jnp.exp(z)
    sm = e / jnp.sum(e, axis=0, keepdims=True)
    out_ref[:D, :] = lax.transpose(emb_ref[...], (1, 0))[:D, :]
    out_ref[D:, :] = sm


_mlp_call = pl.pallas_call(
    _mlp_body,
    grid=(B // _TB,),
    in_specs=[
        pl.BlockSpec((7, _TB), lambda i: (0, i)),
        pl.BlockSpec((_TB, _DP), lambda i: (i, 0)),
        pl.BlockSpec((H1, 7), lambda i: (0, 0)),
        pl.BlockSpec((H1, 1), lambda i: (0, 0)),
        pl.BlockSpec((H2, H1), lambda i: (0, 0)),
        pl.BlockSpec((H2, 1), lambda i: (0, 0)),
        pl.BlockSpec((H3, H2), lambda i: (0, 0)),
        pl.BlockSpec((H3, 1), lambda i: (0, 0)),
    ],
    out_specs=pl.BlockSpec((OUT_D, _TB), lambda i: (0, i)),
    out_shape=jax.ShapeDtypeStruct((OUT_D, B), jnp.float32),
    compiler_params=pltpu.CompilerParams(
        dimension_semantics=("arbitrary",),
    ),
)


def kernel(anime_id, genre, type, episodes, general_rating, members, user_id,
           user_rating, table, W1, b1, W2, b2, W3, b3):
    idx = genre.astype(jnp.int32)
    featsT = jnp.stack(
        [anime_id, type, episodes, general_rating, members, user_id,
         user_rating], axis=0)
    w1t = jnp.swapaxes(W1, 0, 1)
    w2t = jnp.swapaxes(W2, 0, 1).astype(jnp.bfloat16)
    w3t = jnp.swapaxes(W3, 0, 1).astype(jnp.bfloat16)

    table_p = _tp_call(jnp.swapaxes(table, 0, 1))
    emb = _sc_gather(table_p, idx)

    out_t = _mlp_call(featsT, emb, w1t, b1.reshape(H1, 1),
                      w2t, b2.reshape(H2, 1),
                      w3t, b3.reshape(H3, 1))
    return jnp.swapaxes(out_t, 0, 1)


# speedup vs baseline: 2.1298x; 1.0077x over previous
import functools

import jax
import jax.numpy as jnp
from jax import lax
from jax.experimental import pallas as pl
from jax.experimental.pallas import tpu as pltpu
from jax.experimental.pallas import tpu_sc as plsc

B = 16384
V = 100000
D = 317
H1, H2, H3 = 1024, 512, 256
OUT_D = D + H3


_NC = 2
_NS = 16
_NW = _NC * _NS
_B_PER_W = B // _NW
_CHUNK = 128
_N_CHUNKS = _B_PER_W // _CHUNK
_DP = 384

_sc_mesh = plsc.VectorSubcoreMesh(core_axis_name="c", subcore_axis_name="s")


@functools.partial(
    pl.kernel,
    mesh=_sc_mesh,
    out_type=jax.ShapeDtypeStruct((B, _DP), jnp.float32),
    scratch_types=[
        pltpu.VMEM((_B_PER_W,), jnp.int32),
        pltpu.VMEM((2, _CHUNK, _DP), jnp.float32),
        pltpu.SemaphoreType.DMA,
        pltpu.SemaphoreType.DMA,
    ],
)
def _sc_gather(table_hbm, idx_hbm, out_hbm, idx_v, rows_v, gsem, wsem):
    wid = lax.axis_index("s") * _NC + lax.axis_index("c")
    base = wid * _B_PER_W
    pltpu.sync_copy(idx_hbm.at[pl.ds(base, _B_PER_W)], idx_v)

    def gather(c, slot):
        ids = idx_v.at[pl.ds(c * _CHUNK, _CHUNK)]
        pltpu.async_copy(table_hbm.at[ids], rows_v.at[slot], gsem)

    def put(c, slot):
        pltpu.async_copy(rows_v.at[slot],
                         out_hbm.at[pl.ds(base + c * _CHUNK, _CHUNK)], wsem)

    gather(0, 0)
    for c in range(_N_CHUNKS):
        slot = c % 2
        pltpu.make_async_copy(table_hbm, rows_v.at[slot], gsem).wait()
        if c >= 1:
            pltpu.make_async_copy(rows_v.at[0], out_hbm.at[pl.ds(0, _CHUNK)],
                                  wsem).wait()
        if c + 1 < _N_CHUNKS:
            gather(c + 1, 1 - slot)
        put(c, slot)
    pltpu.make_async_copy(rows_v.at[0], out_hbm.at[pl.ds(0, _CHUNK)],
                          wsem).wait()


_CB = 8192


def _tp_body(t_ref, out_ref):
    out_ref[:, :D] = lax.transpose(t_ref[...], (1, 0))


_tp_call = pl.pallas_call(
    _tp_body,
    grid=((V + _CB - 1) // _CB,),
    in_specs=[pl.BlockSpec((D, _CB), lambda i: (0, i))],
    out_specs=pl.BlockSpec((_CB, _DP), lambda i: (i, 0)),
    out_shape=jax.ShapeDtypeStruct((V, _DP), jnp.float32),
    compiler_params=pltpu.CompilerParams(
        dimension_semantics=("arbitrary",),
    ),
)


_TB = 2048


def _mlp_body(featsT_ref, emb_ref, w1t_ref, b1_ref, w2t_ref, b2_ref, w3t_ref,
              b3_ref, out_ref):
    f = featsT_ref[...]
    h = jnp.dot(w1t_ref[...], f, preferred_element_type=jnp.float32)
    h = jnp.maximum(h + b1_ref[...], 0.0)
    h = jnp.dot(w2t_ref[...], h.astype(jnp.bfloat16),
                preferred_element_type=jnp.float32)
    h = jnp.maximum(h + b2_ref[...], 0.0)
    z = jnp.dot(w3t_ref[...], h.astype(jnp.bfloat16),
                preferred_element_type=jnp.float32)
    z = z + b3_ref[...]
    z = z - jnp.max(z, axis=0, keepdims=True)
    e = jnp.exp(z)
    sm = e / jnp.sum(e, axis=0, keepdims=True)
    out_ref[:D, :] = lax.transpose(emb_ref[...], (1, 0))[:D, :]
    out_ref[D:, :] = sm


_mlp_call = pl.pallas_call(
    _mlp_body,
    grid=(B // _TB,),
    in_specs=[
        pl.BlockSpec((7, _TB), lambda i: (0, i)),
        pl.BlockSpec((_TB, _DP), lambda i: (i, 0)),
        pl.BlockSpec((H1, 7), lambda i: (0, 0)),
        pl.BlockSpec((H1, 1), lambda i: (0, 0)),
        pl.BlockSpec((H2, H1), lambda i: (0, 0)),
        pl.BlockSpec((H2, 1), lambda i: (0, 0)),
        pl.BlockSpec((H3, H2), lambda i: (0, 0)),
        pl.BlockSpec((H3, 1), lambda i: (0, 0)),
    ],
    out_specs=pl.BlockSpec((OUT_D, _TB), lambda i: (0, i)),
    out_shape=jax.ShapeDtypeStruct((OUT_D, B), jnp.float32),
    compiler_params=pltpu.CompilerParams(
        dimension_semantics=("arbitrary",),
    ),
)


def kernel(anime_id, genre, type, episodes, general_rating, members, user_id,
           user_rating, table, W1, b1, W2, b2, W3, b3):
    idx = genre.astype(jnp.int32)
    featsT = jnp.stack(
        [anime_id, type, episodes, general_rating, members, user_id,
         user_rating], axis=0)
    w1t = jnp.swapaxes(W1, 0, 1)
    w2t = jnp.swapaxes(W2, 0, 1).astype(jnp.bfloat16)
    w3t = jnp.swapaxes(W3, 0, 1).astype(jnp.bfloat16)

    table_p = _tp_call(jnp.swapaxes(table, 0, 1))
    emb = _sc_gather(table_p, idx)

    out_t = _mlp_call(featsT, emb, w1t, b1.reshape(H1, 1),
                      w2t, b2.reshape(H2, 1),
                      w3t, b3.reshape(H3, 1))
    return jnp.swapaxes(out_t, 0, 1)
